# baseline (speedup 1.0000x reference)
import jax
import jax.numpy as jnp
from jax import lax
from jax.experimental import pallas as pl
from jax.experimental.pallas import tpu as pltpu

_MUX_IN = 32768
_MUX_OUT = 16


def _topk_body(logits_ref, gn_ref, idx_ref):
    x = logits_ref[...] + gn_ref[...]
    iota = lax.broadcasted_iota(jnp.int32, x.shape, 1)
    cols = []
    for _ in range(_MUX_OUT):
        idx = jnp.argmax(x, axis=1).astype(jnp.int32)[:, None]
        cols.append(idx)
        x = jnp.where(iota == idx, -jnp.inf, x)
    idx_ref[...] = jnp.concatenate(cols, axis=1)


_R = _MUX_IN // 128


def _onehot_body(idx_sref, out_ref):
    i = pl.program_id(0)
    out_ref[...] = jnp.zeros((_MUX_OUT, _R, 128), jnp.float32)
    lane = lax.broadcasted_iota(jnp.int32, (1, 1, 128), 2)
    for j in range(_MUX_OUT):
        idx = idx_sref[i * _MUX_OUT + j]
        s = idx // 128
        l = idx - s * 128
        out_ref[pl.ds(j, 1), pl.ds(s, 1), :] = (lane == l).astype(jnp.float32)


def kernel(logits, inpData):
    BS = inpData.shape[0]
    u = jax.random.uniform(
        jax.random.key(42), (BS, _MUX_IN), minval=0.0, maxval=1.0,
        dtype=jnp.float32)
    gn = -jnp.log(-jnp.log(u + 1e-20) + 1e-20)

    topk_idx = pl.pallas_call(
        _topk_body,
        grid=(1,),
        in_specs=[
            pl.BlockSpec((1, _MUX_IN), lambda i: (0, 0)),
            pl.BlockSpec((BS, _MUX_IN), lambda i: (0, 0)),
        ],
        out_specs=pl.BlockSpec((BS, _MUX_OUT), lambda i: (0, 0)),
        out_shape=jax.ShapeDtypeStruct((BS, _MUX_OUT), jnp.int32),
    )(logits, gn)

    out = pl.pallas_call(
        _onehot_body,
        grid_spec=pltpu.PrefetchScalarGridSpec(
            num_scalar_prefetch=1,
            grid=(BS,),
            in_specs=[],
            out_specs=pl.BlockSpec((_MUX_OUT, _R, 128),
                                   lambda i, idx_ref: (i, 0, 0)),
        ),
        out_shape=jax.ShapeDtypeStruct((BS * _MUX_OUT, _R, 128), jnp.float32),
    )(topk_idx.reshape(-1))
    return out.reshape(BS, _MUX_OUT, _MUX_IN)


# speedup vs baseline: 2.0640x; 2.0640x over previous
import jax
import jax.numpy as jnp
from jax import lax
from jax.experimental import pallas as pl
from jax.experimental.pallas import tpu as pltpu

_MUX_IN = 32768
_MUX_OUT = 16


def _topk_body(logits_ref, gn_ref, idx_ref):
    x = logits_ref[...] + gn_ref[...]
    iota = lax.broadcasted_iota(jnp.int32, x.shape, 1)
    cols = []
    for _ in range(_MUX_OUT):
        idx = jnp.argmax(x, axis=1).astype(jnp.int32)[:, None]
        cols.append(idx)
        x = jnp.where(iota == idx, -jnp.inf, x)
    idx_ref[...] = jnp.concatenate(cols, axis=1)


def _onehot_body(idx_sref, out_ref):
    i = pl.program_id(0)
    idxv = jnp.array([idx_sref[i * _MUX_OUT + j] for j in range(_MUX_OUT)],
                     dtype=jnp.int32).reshape(_MUX_OUT, 1)
    iota = lax.broadcasted_iota(jnp.int32, (_MUX_OUT, _MUX_IN), 1)
    out_ref[...] = (iota == idxv).astype(jnp.float32)


def kernel(logits, inpData):
    BS = inpData.shape[0]
    u = jax.random.uniform(
        jax.random.key(42), (BS, _MUX_IN), minval=0.0, maxval=1.0,
        dtype=jnp.float32)
    gn = -jnp.log(-jnp.log(u + 1e-20) + 1e-20)

    topk_idx = pl.pallas_call(
        _topk_body,
        grid=(1,),
        in_specs=[
            pl.BlockSpec((1, _MUX_IN), lambda i: (0, 0)),
            pl.BlockSpec((BS, _MUX_IN), lambda i: (0, 0)),
        ],
        out_specs=pl.BlockSpec((BS, _MUX_OUT), lambda i: (0, 0)),
        out_shape=jax.ShapeDtypeStruct((BS, _MUX_OUT), jnp.int32),
    )(logits, gn)

    out = pl.pallas_call(
        _onehot_body,
        grid_spec=pltpu.PrefetchScalarGridSpec(
            num_scalar_prefetch=1,
            grid=(BS,),
            in_specs=[],
            out_specs=pl.BlockSpec((_MUX_OUT, _MUX_IN),
                                   lambda i, idx_ref: (i, 0)),
        ),
        out_shape=jax.ShapeDtypeStruct((BS * _MUX_OUT, _MUX_IN), jnp.float32),
    )(topk_idx.reshape(-1))
    return out.reshape(BS, _MUX_OUT, _MUX_IN)


# speedup vs baseline: 2.1201x; 1.0272x over previous
import jax
import jax.numpy as jnp
from jax import lax
from jax.experimental import pallas as pl
from jax.experimental.pallas import tpu as pltpu

_MUX_IN = 32768
_MUX_OUT = 16


def _fused_body(logits_ref, gn_ref, out_ref, idx_scratch):
    i = pl.program_id(0)

    @pl.when(i == 0)
    def _compute_topk():
        x = logits_ref[...] + gn_ref[...]
        iota = lax.broadcasted_iota(jnp.int32, x.shape, 1)
        cols = []
        for _ in range(_MUX_OUT):
            idx = jnp.argmax(x, axis=1).astype(jnp.int32)[:, None]
            cols.append(idx)
            x = jnp.where(iota == idx, -jnp.inf, x)
        idx_scratch[...] = jnp.concatenate(cols, axis=1)

    row = idx_scratch[pl.ds(i, 1), :]
    idxv = jnp.concatenate(
        [row[:, j:j + 1] for j in range(_MUX_OUT)], axis=0)
    iota2 = lax.broadcasted_iota(jnp.int32, (_MUX_OUT, _MUX_IN), 1)
    out_ref[...] = (iota2 == idxv).astype(jnp.float32)


def kernel(logits, inpData):
    BS = inpData.shape[0]
    u = jax.random.uniform(
        jax.random.key(42), (BS, _MUX_IN), minval=0.0, maxval=1.0,
        dtype=jnp.float32)
    gn = -jnp.log(-jnp.log(u + 1e-20) + 1e-20)

    out = pl.pallas_call(
        _fused_body,
        grid=(BS,),
        in_specs=[
            pl.BlockSpec((1, _MUX_IN), lambda i: (0, 0)),
            pl.BlockSpec((BS, _MUX_IN), lambda i: (0, 0)),
        ],
        out_specs=pl.BlockSpec((_MUX_OUT, _MUX_IN), lambda i: (i, 0)),
        out_shape=jax.ShapeDtypeStruct((BS * _MUX_OUT, _MUX_IN), jnp.float32),
        scratch_shapes=[pltpu.VMEM((BS, _MUX_OUT), jnp.int32)],
    )(logits, gn)
    return out.reshape(BS, _MUX_OUT, _MUX_IN)


# speedup vs baseline: 2.1672x; 1.0222x over previous
import jax
import jax.numpy as jnp
from jax import lax
from jax.experimental import pallas as pl
from jax.experimental.pallas import tpu as pltpu

_MUX_IN = 32768
_MUX_OUT = 16


def _fused_body(logits_ref, gn_ref, out_ref, idx_scratch):
    i = pl.program_id(0)

    @pl.when(i == 0)
    def _compute_topk():
        x = logits_ref[...] + gn_ref[...]
        iota = lax.broadcasted_iota(jnp.int32, x.shape, 1)
        cols = []
        for _ in range(_MUX_OUT):
            idx = jnp.argmax(x, axis=1).astype(jnp.int32)[:, None]
            cols.append(idx)
            x = jnp.where(iota == idx, -jnp.inf, x)
        idx_scratch[...] = jnp.concatenate(cols, axis=1)

    row = idx_scratch[pl.ds(i, 1), :]
    idxv = row.reshape(_MUX_OUT, 1)
    iota2 = lax.broadcasted_iota(jnp.int32, (_MUX_OUT, _MUX_IN), 1)
    out_ref[...] = (iota2 == idxv).astype(jnp.float32)


def kernel(logits, inpData):
    BS = inpData.shape[0]
    u = jax.random.uniform(
        jax.random.key(42), (BS, _MUX_IN), minval=0.0, maxval=1.0,
        dtype=jnp.float32)
    gn = -jnp.log(-jnp.log(u + 1e-20) + 1e-20)

    out = pl.pallas_call(
        _fused_body,
        grid=(BS,),
        in_specs=[
            pl.BlockSpec((1, _MUX_IN), lambda i: (0, 0)),
            pl.BlockSpec((BS, _MUX_IN), lambda i: (0, 0)),
        ],
        out_specs=pl.BlockSpec((_MUX_OUT, _MUX_IN), lambda i: (i, 0)),
        out_shape=jax.ShapeDtypeStruct((BS * _MUX_OUT, _MUX_IN), jnp.float32),
        scratch_shapes=[pltpu.VMEM((BS, _MUX_OUT), jnp.int32)],
    )(logits, gn)
    return out.reshape(BS, _MUX_OUT, _MUX_IN)
